# baseline (speedup 1.0000x reference)
import functools
import math

import jax
import jax.numpy as jnp
from jax import lax
from jax.experimental import pallas as pl
from jax.experimental.pallas import tpu as pltpu
from jax.experimental.pallas import tpu_sc as plsc

B, T, C, H = 2, 2048, 2048, 16
DH = C // H
KCAP = T // 2
DFF = 4 * C
R = B * KCAP

_BF = jnp.bfloat16
_F32 = jnp.float32


_NC, _NS = 2, 16
_NW = _NC * _NS


def _sc_gather(table, idxs, n_rows, chunk=32):
    d = table.shape[1]
    per_w = n_rows // _NW
    n_chunks = per_w // chunk
    mesh = plsc.VectorSubcoreMesh(
        core_axis_name="c", subcore_axis_name="s",
        num_cores=_NC, num_subcores=_NS)

    @functools.partial(
        pl.kernel,
        out_type=jax.ShapeDtypeStruct((n_rows, d), table.dtype),
        mesh=mesh,
        scratch_types=[
            pltpu.VMEM((chunk,), jnp.int32),
            pltpu.VMEM((chunk, d), table.dtype),
            pltpu.SemaphoreType.DMA,
        ],
    )
    def k(table_hbm, idx_hbm, out_hbm, idx_v, rows_v, sem):
        wid = lax.axis_index("s") * _NC + lax.axis_index("c")
        for j in range(n_chunks):
            base = wid * per_w + j * chunk
            pltpu.sync_copy(idx_hbm.at[pl.ds(base, chunk)], idx_v)
            pltpu.async_copy(table_hbm.at[idx_v], rows_v, sem).wait()
            pltpu.sync_copy(rows_v, out_hbm.at[pl.ds(base, chunk)])

    return k(table, idxs)


def _sc_combine(x_flat, final, sidx, uidx):
    chunk = 32
    per_w = R // _NW
    n_chunks = per_w // chunk
    mesh = plsc.VectorSubcoreMesh(
        core_axis_name="c", subcore_axis_name="s",
        num_cores=_NC, num_subcores=_NS)

    @functools.partial(
        pl.kernel,
        out_type=jax.ShapeDtypeStruct((B * T, C), jnp.float32),
        mesh=mesh,
        scratch_types=[
            pltpu.VMEM((chunk,), jnp.int32),
            pltpu.VMEM((chunk, C), jnp.float32),
            pltpu.SemaphoreType.DMA,
        ],
    )
    def k(x_hbm, final_hbm, sidx_hbm, uidx_hbm, out_hbm, idx_v, rows_v, sem):
        wid = lax.axis_index("s") * _NC + lax.axis_index("c")
        for j in range(n_chunks):
            base = wid * per_w + j * chunk
            pltpu.sync_copy(sidx_hbm.at[pl.ds(base, chunk)], idx_v)
            pltpu.sync_copy(final_hbm.at[pl.ds(base, chunk)], rows_v)
            pltpu.async_copy(rows_v, out_hbm.at[idx_v], sem).wait()
        for j in range(n_chunks):
            base = wid * per_w + j * chunk
            pltpu.sync_copy(uidx_hbm.at[pl.ds(base, chunk)], idx_v)
            pltpu.async_copy(x_hbm.at[idx_v], rows_v, sem).wait()
            pltpu.async_copy(rows_v, out_hbm.at[idx_v], sem).wait()

    return k(x_flat, final, sidx, uidx)


def _router_body(x_ref, w_ref, o_ref):
    i = pl.program_id(0)
    lg = lax.dot_general(x_ref[...].astype(_BF), w_ref[...].astype(_BF),
                         (((1,), (0,)), ((), ())),
                         preferred_element_type=_F32)
    o_ref[pl.ds(i * 512, 512)] = lg[:, 0]


def _router_logits(x_flat, w_router):
    return pl.pallas_call(
        _router_body,
        grid=(B * T // 512,),
        in_specs=[
            pl.BlockSpec((512, C), lambda i: (i, 0)),
            pl.BlockSpec((C, 1), lambda i: (0, 0)),
        ],
        out_specs=pl.BlockSpec((B * T,), lambda i: (0,)),
        out_shape=jax.ShapeDtypeStruct((B * T,), _F32),
    )(x_flat, w_router.reshape(C, 1))


def _rms_body(x_ref, w_ref, o_ref):
    x = x_ref[...]
    ms = jnp.mean(x * x, axis=-1, keepdims=True)
    o_ref[...] = (x * lax.rsqrt(ms + 1e-6) * w_ref[...][None, :]).astype(_BF)


def _rmsnorm_bf16(x, w):
    return pl.pallas_call(
        _rms_body,
        grid=(R // 256,),
        in_specs=[
            pl.BlockSpec((256, C), lambda i: (i, 0)),
            pl.BlockSpec((C,), lambda i: (0,)),
        ],
        out_specs=pl.BlockSpec((256, C), lambda i: (i, 0)),
        out_shape=jax.ShapeDtypeStruct((R, C), _BF),
    )(x, w)


def _qkv_body(h_ref, wq_ref, wk_ref, wv_ref, q_ref, k_ref, v_ref):
    h = h_ref[...]
    for wref, oref in ((wq_ref, q_ref), (wk_ref, k_ref), (wv_ref, v_ref)):
        w = wref[...].astype(_BF)
        acc = lax.dot_general(h, w, (((1,), (0,)), ((), ())),
                              preferred_element_type=_F32)
        oref[...] = acc.astype(_BF)


def _qkv(h, Wq, Wk, Wv):
    bn = 512
    wspec = pl.BlockSpec((C, bn), lambda n: (0, n))
    ospec = pl.BlockSpec((R, bn), lambda n: (0, n))
    oshape = jax.ShapeDtypeStruct((R, C), _BF)
    return pl.pallas_call(
        _qkv_body,
        grid=(C // bn,),
        in_specs=[pl.BlockSpec((R, C), lambda n: (0, 0)), wspec, wspec, wspec],
        out_specs=(ospec, ospec, ospec),
        out_shape=(oshape, oshape, oshape),
    )(h, Wq, Wk, Wv)


_BQ = 512
_NQ = KCAP // _BQ


def _att_body(q_ref, k_ref, v_ref, o_ref, m_s, l_s, acc_s):
    qi = pl.program_id(2)
    kj = pl.program_id(3)

    @pl.when(kj == 0)
    def _():
        m_s[...] = jnp.full_like(m_s, -1e30)
        l_s[...] = jnp.zeros_like(l_s)
        acc_s[...] = jnp.zeros_like(acc_s)

    @pl.when(kj <= qi)
    def _():
        s = lax.dot_general(q_ref[...], k_ref[...], (((1,), (1,)), ((), ())),
                            preferred_element_type=_F32)
        s = s * (1.0 / math.sqrt(DH))
        row = qi * _BQ + lax.broadcasted_iota(jnp.int32, (_BQ, _BQ), 0)
        col = kj * _BQ + lax.broadcasted_iota(jnp.int32, (_BQ, _BQ), 1)
        s = jnp.where(row >= col, s, -1e30)
        m_old = m_s[...]
        m_new = jnp.maximum(m_old, jnp.max(s, axis=1))
        corr = jnp.exp(m_old - m_new)
        p = jnp.exp(s - m_new[:, None])
        l_s[...] = l_s[...] * corr + jnp.sum(p, axis=1)
        acc_s[...] = acc_s[...] * corr[:, None] + lax.dot_general(
            p.astype(_BF), v_ref[...], (((1,), (0,)), ((), ())),
            preferred_element_type=_F32)
        m_s[...] = m_new

    @pl.when(kj == pl.num_programs(3) - 1)
    def _():
        o_ref[...] = (acc_s[...] / l_s[...][:, None]).astype(_BF)


def _attention(q, k, v):
    qspec = pl.BlockSpec((_BQ, DH), lambda b, h, qi, kj: (_NQ * b + qi, h))
    kspec = pl.BlockSpec((_BQ, DH), lambda b, h, qi, kj: (_NQ * b + kj, h))
    return pl.pallas_call(
        _att_body,
        grid=(B, H, _NQ, _NQ),
        in_specs=[qspec, kspec, kspec],
        out_specs=qspec,
        out_shape=jax.ShapeDtypeStruct((R, C), _BF),
        scratch_shapes=[
            pltpu.VMEM((_BQ,), _F32),
            pltpu.VMEM((_BQ,), _F32),
            pltpu.VMEM((_BQ, DH), _F32),
        ],
    )(q, k, v)


def _wo_body(a_ref, wo_ref, sel_ref, gl_ref, ln2_ref, h2_ref, part_ref, acc_ref):
    kk = pl.program_id(1)
    nk = pl.num_programs(1)

    @pl.when(kk == 0)
    def _():
        acc_ref[...] = jnp.zeros_like(acc_ref)

    acc_ref[...] += lax.dot_general(
        a_ref[...], wo_ref[...].astype(_BF), (((1,), (0,)), ((), ())),
        preferred_element_type=_F32)

    @pl.when(kk == nk - 1)
    def _():
        y2 = acc_ref[...]
        selb = sel_ref[...]
        sel2 = selb + y2
        ms = jnp.mean(sel2 * sel2, axis=-1, keepdims=True)
        h2_ref[...] = (sel2 * lax.rsqrt(ms + 1e-6) * ln2_ref[...][None, :]).astype(_BF)
        gate = jax.nn.sigmoid(gl_ref[...])[:, None]
        part_ref[...] = selb + gate * y2


def _wo_norm(attnout, Wo, sel, gl, ln2_w):
    bm, bk = 1024, 256
    return pl.pallas_call(
        _wo_body,
        grid=(R // bm, C // bk),
        in_specs=[
            pl.BlockSpec((bm, bk), lambda m, k: (m, k)),
            pl.BlockSpec((bk, C), lambda m, k: (k, 0)),
            pl.BlockSpec((bm, C), lambda m, k: (m, 0)),
            pl.BlockSpec((bm,), lambda m, k: (m,)),
            pl.BlockSpec((C,), lambda m, k: (0,)),
        ],
        out_specs=(
            pl.BlockSpec((bm, C), lambda m, k: (m, 0)),
            pl.BlockSpec((bm, C), lambda m, k: (m, 0)),
        ),
        out_shape=(
            jax.ShapeDtypeStruct((R, C), _BF),
            jax.ShapeDtypeStruct((R, C), _F32),
        ),
        scratch_shapes=[pltpu.VMEM((bm, C), _F32)],
    )(attnout, Wo, sel, gl, ln2_w)


def _mlp1_body(h_ref, w_ref, o_ref):
    acc = lax.dot_general(h_ref[...], w_ref[...].astype(_BF),
                          (((1,), (0,)), ((), ())), preferred_element_type=_F32)
    o_ref[...] = jax.nn.gelu(acc, approximate=True).astype(_BF)


def _mlp1(h2, W1):
    bn = 512
    return pl.pallas_call(
        _mlp1_body,
        grid=(DFF // bn,),
        in_specs=[
            pl.BlockSpec((R, C), lambda n: (0, 0)),
            pl.BlockSpec((C, bn), lambda n: (0, n)),
        ],
        out_specs=pl.BlockSpec((R, bn), lambda n: (0, n)),
        out_shape=jax.ShapeDtypeStruct((R, DFF), _BF),
    )(h2, W1)


def _mlp2_body(g_ref, w_ref, part_ref, gl_ref, o_ref, acc_ref):
    kk = pl.program_id(1)
    nk = pl.num_programs(1)

    @pl.when(kk == 0)
    def _():
        acc_ref[...] = jnp.zeros_like(acc_ref)

    acc_ref[...] += lax.dot_general(
        g_ref[...], w_ref[...].astype(_BF), (((1,), (0,)), ((), ())),
        preferred_element_type=_F32)

    @pl.when(kk == nk - 1)
    def _():
        gate = jax.nn.sigmoid(gl_ref[...])[:, None]
        o_ref[...] = part_ref[...] + gate * acc_ref[...]


def _mlp2_final(g, W2, partial, gl):
    bn, bk = 1024, 512
    return pl.pallas_call(
        _mlp2_body,
        grid=(C // bn, DFF // bk),
        in_specs=[
            pl.BlockSpec((R, bk), lambda n, k: (0, k)),
            pl.BlockSpec((bk, bn), lambda n, k: (k, n)),
            pl.BlockSpec((R, bn), lambda n, k: (0, n)),
            pl.BlockSpec((R,), lambda n, k: (0,)),
        ],
        out_specs=pl.BlockSpec((R, bn), lambda n, k: (0, n)),
        out_shape=jax.ShapeDtypeStruct((R, C), _F32),
        scratch_shapes=[pltpu.VMEM((R, bn), _F32)],
    )(g, W2, partial, gl)


def kernel(x, w_router, ln1_w, ln2_w, Wq, Wk, Wv, Wo, W1, W2):
    x_flat = x.reshape(B * T, C)

    logits = _router_logits(x_flat, w_router).reshape(B, T)
    _, idx = lax.top_k(logits, KCAP)
    idx = jnp.sort(idx, axis=1)
    gl = jnp.take_along_axis(logits, idx, axis=1).reshape(R)
    idx_flat = (idx + jnp.arange(B, dtype=idx.dtype)[:, None] * T).reshape(R)
    idx_flat = idx_flat.astype(jnp.int32)

    sel = _sc_gather(x_flat, idx_flat, R)

    h1 = _rmsnorm_bf16(sel, ln1_w)
    q, k, v = _qkv(h1, Wq, Wk, Wv)
    attnout = _attention(q, k, v)
    h2, partial = _wo_norm(attnout, Wo, sel, gl, ln2_w)
    g = _mlp1(h2, W1)
    final = _mlp2_final(g, W2, partial, gl)

    flags = jnp.zeros((B * T,), jnp.int32).at[idx_flat].set(1)
    slot = jnp.cumsum(1 - flags) - 1
    tgt = jnp.where(flags == 0, slot, R)
    uidx = jnp.zeros((R,), jnp.int32).at[tgt].set(
        jnp.arange(B * T, dtype=jnp.int32), mode="drop")
    out = _sc_combine(x_flat, final, idx_flat, uidx)
    return out.reshape(B, T, C)


# speedup vs baseline: 1.2093x; 1.2093x over previous
import functools
import math

import jax
import jax.numpy as jnp
from jax import lax
from jax.experimental import pallas as pl
from jax.experimental.pallas import tpu as pltpu
from jax.experimental.pallas import tpu_sc as plsc

B, T, C, H = 2, 2048, 2048, 16
DH = C // H
KCAP = T // 2
DFF = 4 * C
R = B * KCAP

_BF = jnp.bfloat16
_F32 = jnp.float32


_NC, _NS = 2, 16
_NW = _NC * _NS


def _sc_gather(table, idxs, n_rows, chunk=32):
    d = table.shape[1]
    per_w = n_rows // _NW
    n_chunks = per_w // chunk
    mesh = plsc.VectorSubcoreMesh(
        core_axis_name="c", subcore_axis_name="s",
        num_cores=_NC, num_subcores=_NS)

    @functools.partial(
        pl.kernel,
        out_type=jax.ShapeDtypeStruct((n_rows, d), table.dtype),
        mesh=mesh,
        scratch_types=[
            pltpu.VMEM((chunk,), jnp.int32),
            pltpu.VMEM((chunk, d), table.dtype),
            pltpu.SemaphoreType.DMA,
        ],
    )
    def k(table_hbm, idx_hbm, out_hbm, idx_v, rows_v, sem):
        wid = lax.axis_index("s") * _NC + lax.axis_index("c")
        for j in range(n_chunks):
            base = wid * per_w + j * chunk
            pltpu.sync_copy(idx_hbm.at[pl.ds(base, chunk)], idx_v)
            pltpu.async_copy(table_hbm.at[idx_v], rows_v, sem).wait()
            pltpu.sync_copy(rows_v, out_hbm.at[pl.ds(base, chunk)])

    return k(table, idxs)


def _sc_combine(x_flat, final, sidx, uidx):
    chunk = 32
    per_w = R // _NW
    n_chunks = per_w // chunk
    mesh = plsc.VectorSubcoreMesh(
        core_axis_name="c", subcore_axis_name="s",
        num_cores=_NC, num_subcores=_NS)

    @functools.partial(
        pl.kernel,
        out_type=jax.ShapeDtypeStruct((B * T, C), jnp.float32),
        mesh=mesh,
        scratch_types=[
            pltpu.VMEM((chunk,), jnp.int32),
            pltpu.VMEM((chunk, C), jnp.float32),
            pltpu.SemaphoreType.DMA,
        ],
    )
    def k(x_hbm, final_hbm, sidx_hbm, uidx_hbm, out_hbm, idx_v, rows_v, sem):
        wid = lax.axis_index("s") * _NC + lax.axis_index("c")
        for j in range(n_chunks):
            base = wid * per_w + j * chunk
            pltpu.sync_copy(sidx_hbm.at[pl.ds(base, chunk)], idx_v)
            pltpu.sync_copy(final_hbm.at[pl.ds(base, chunk)], rows_v)
            pltpu.async_copy(rows_v, out_hbm.at[idx_v], sem).wait()
        for j in range(n_chunks):
            base = wid * per_w + j * chunk
            pltpu.sync_copy(uidx_hbm.at[pl.ds(base, chunk)], idx_v)
            pltpu.async_copy(x_hbm.at[idx_v], rows_v, sem).wait()
            pltpu.async_copy(rows_v, out_hbm.at[idx_v], sem).wait()

    return k(x_flat, final, sidx, uidx)


def _router_body(x_ref, w_ref, o_ref):
    i = pl.program_id(0)
    lg = lax.dot_general(x_ref[...].astype(_BF), w_ref[...].astype(_BF),
                         (((1,), (0,)), ((), ())),
                         preferred_element_type=_F32)
    o_ref[pl.ds(i * 512, 512)] = lg[:, 0]


def _router_logits(x_flat, w_router):
    return pl.pallas_call(
        _router_body,
        grid=(B * T // 512,),
        in_specs=[
            pl.BlockSpec((512, C), lambda i: (i, 0)),
            pl.BlockSpec((C, 1), lambda i: (0, 0)),
        ],
        out_specs=pl.BlockSpec((B * T,), lambda i: (0,)),
        out_shape=jax.ShapeDtypeStruct((B * T,), _F32),
    )(x_flat, w_router.reshape(C, 1))


def _rms_body(x_ref, w_ref, o_ref):
    x = x_ref[...]
    ms = jnp.mean(x * x, axis=-1, keepdims=True)
    o_ref[...] = (x * lax.rsqrt(ms + 1e-6) * w_ref[...][None, :]).astype(_BF)


def _rmsnorm_bf16(x, w):
    return pl.pallas_call(
        _rms_body,
        grid=(R // 256,),
        in_specs=[
            pl.BlockSpec((256, C), lambda i: (i, 0)),
            pl.BlockSpec((C,), lambda i: (0,)),
        ],
        out_specs=pl.BlockSpec((256, C), lambda i: (i, 0)),
        out_shape=jax.ShapeDtypeStruct((R, C), _BF),
    )(x, w)


def _qkv_body(h_ref, wq_ref, wk_ref, wv_ref, q_ref, k_ref, v_ref):
    h = h_ref[...]
    for wref, oref in ((wq_ref, q_ref), (wk_ref, k_ref), (wv_ref, v_ref)):
        w = wref[...].astype(_BF)
        acc = lax.dot_general(h, w, (((1,), (0,)), ((), ())),
                              preferred_element_type=_F32)
        oref[...] = acc.astype(_BF)


def _qkv(h, Wq, Wk, Wv):
    bn = 512
    wspec = pl.BlockSpec((C, bn), lambda n: (0, n))
    ospec = pl.BlockSpec((R, bn), lambda n: (0, n))
    oshape = jax.ShapeDtypeStruct((R, C), _BF)
    return pl.pallas_call(
        _qkv_body,
        grid=(C // bn,),
        in_specs=[pl.BlockSpec((R, C), lambda n: (0, 0)), wspec, wspec, wspec],
        out_specs=(ospec, ospec, ospec),
        out_shape=(oshape, oshape, oshape),
    )(h, Wq, Wk, Wv)


def _att_body(q_ref, k_ref, v_ref, o_ref):
    q = q_ref[...]
    s = lax.dot_general(q, k_ref[...], (((1,), (1,)), ((), ())),
                        preferred_element_type=_F32)
    s = s * (1.0 / math.sqrt(DH))
    row = lax.broadcasted_iota(jnp.int32, (KCAP, KCAP), 0)
    col = lax.broadcasted_iota(jnp.int32, (KCAP, KCAP), 1)
    s = jnp.where(row >= col, s, -jnp.finfo(_F32).max)
    m = jnp.max(s, axis=-1, keepdims=True)
    e = jnp.exp(s - m)
    p = e / jnp.sum(e, axis=-1, keepdims=True)
    y = lax.dot_general(p.astype(_BF), v_ref[...], (((1,), (0,)), ((), ())),
                        preferred_element_type=_F32)
    o_ref[...] = y.astype(_BF)


def _attention(q, k, v):
    spec = pl.BlockSpec((KCAP, DH), lambda b, h: (b, h))
    return pl.pallas_call(
        _att_body,
        grid=(B, H),
        in_specs=[spec, spec, spec],
        out_specs=spec,
        out_shape=jax.ShapeDtypeStruct((R, C), _BF),
    )(q, k, v)


def _wo_body(a_ref, wo_ref, sel_ref, gl_ref, ln2_ref, h2_ref, part_ref, acc_ref):
    kk = pl.program_id(1)
    nk = pl.num_programs(1)

    @pl.when(kk == 0)
    def _():
        acc_ref[...] = jnp.zeros_like(acc_ref)

    acc_ref[...] += lax.dot_general(
        a_ref[...], wo_ref[...].astype(_BF), (((1,), (0,)), ((), ())),
        preferred_element_type=_F32)

    @pl.when(kk == nk - 1)
    def _():
        y2 = acc_ref[...]
        selb = sel_ref[...]
        sel2 = selb + y2
        ms = jnp.mean(sel2 * sel2, axis=-1, keepdims=True)
        h2_ref[...] = (sel2 * lax.rsqrt(ms + 1e-6) * ln2_ref[...][None, :]).astype(_BF)
        gate = jax.nn.sigmoid(gl_ref[...])[:, None]
        part_ref[...] = selb + gate * y2


def _wo_norm(attnout, Wo, sel, gl, ln2_w):
    bm, bk = 1024, 256
    return pl.pallas_call(
        _wo_body,
        grid=(R // bm, C // bk),
        in_specs=[
            pl.BlockSpec((bm, bk), lambda m, k: (m, k)),
            pl.BlockSpec((bk, C), lambda m, k: (k, 0)),
            pl.BlockSpec((bm, C), lambda m, k: (m, 0)),
            pl.BlockSpec((bm,), lambda m, k: (m,)),
            pl.BlockSpec((C,), lambda m, k: (0,)),
        ],
        out_specs=(
            pl.BlockSpec((bm, C), lambda m, k: (m, 0)),
            pl.BlockSpec((bm, C), lambda m, k: (m, 0)),
        ),
        out_shape=(
            jax.ShapeDtypeStruct((R, C), _BF),
            jax.ShapeDtypeStruct((R, C), _F32),
        ),
        scratch_shapes=[pltpu.VMEM((bm, C), _F32)],
    )(attnout, Wo, sel, gl, ln2_w)


def _mlp1_body(h_ref, w_ref, o_ref):
    acc = lax.dot_general(h_ref[...], w_ref[...].astype(_BF),
                          (((1,), (0,)), ((), ())), preferred_element_type=_F32)
    o_ref[...] = jax.nn.gelu(acc, approximate=True).astype(_BF)


def _mlp1(h2, W1):
    bn = 512
    return pl.pallas_call(
        _mlp1_body,
        grid=(DFF // bn,),
        in_specs=[
            pl.BlockSpec((R, C), lambda n: (0, 0)),
            pl.BlockSpec((C, bn), lambda n: (0, n)),
        ],
        out_specs=pl.BlockSpec((R, bn), lambda n: (0, n)),
        out_shape=jax.ShapeDtypeStruct((R, DFF), _BF),
    )(h2, W1)


def _mlp2_body(g_ref, w_ref, part_ref, gl_ref, o_ref, acc_ref):
    kk = pl.program_id(1)
    nk = pl.num_programs(1)

    @pl.when(kk == 0)
    def _():
        acc_ref[...] = jnp.zeros_like(acc_ref)

    acc_ref[...] += lax.dot_general(
        g_ref[...], w_ref[...].astype(_BF), (((1,), (0,)), ((), ())),
        preferred_element_type=_F32)

    @pl.when(kk == nk - 1)
    def _():
        gate = jax.nn.sigmoid(gl_ref[...])[:, None]
        o_ref[...] = part_ref[...] + gate * acc_ref[...]


def _mlp2_final(g, W2, partial, gl):
    bn, bk = 1024, 512
    return pl.pallas_call(
        _mlp2_body,
        grid=(C // bn, DFF // bk),
        in_specs=[
            pl.BlockSpec((R, bk), lambda n, k: (0, k)),
            pl.BlockSpec((bk, bn), lambda n, k: (k, n)),
            pl.BlockSpec((R, bn), lambda n, k: (0, n)),
            pl.BlockSpec((R,), lambda n, k: (0,)),
        ],
        out_specs=pl.BlockSpec((R, bn), lambda n, k: (0, n)),
        out_shape=jax.ShapeDtypeStruct((R, C), _F32),
        scratch_shapes=[pltpu.VMEM((R, bn), _F32)],
    )(g, W2, partial, gl)


def kernel(x, w_router, ln1_w, ln2_w, Wq, Wk, Wv, Wo, W1, W2):
    x_flat = x.reshape(B * T, C)

    logits = _router_logits(x_flat, w_router).reshape(B, T)
    _, idx = lax.top_k(logits, KCAP)
    idx = jnp.sort(idx, axis=1)
    gl = jnp.take_along_axis(logits, idx, axis=1).reshape(R)
    idx_flat = (idx + jnp.arange(B, dtype=idx.dtype)[:, None] * T).reshape(R)
    idx_flat = idx_flat.astype(jnp.int32)

    sel = _sc_gather(x_flat, idx_flat, R)

    h1 = _rmsnorm_bf16(sel, ln1_w)
    q, k, v = _qkv(h1, Wq, Wk, Wv)
    attnout = _attention(q, k, v)
    h2, partial = _wo_norm(attnout, Wo, sel, gl, ln2_w)
    g = _mlp1(h2, W1)
    final = _mlp2_final(g, W2, partial, gl)

    flags = jnp.zeros((B * T,), jnp.int32).at[idx_flat].set(1)
    slot = jnp.cumsum(1 - flags) - 1
    tgt = jnp.where(flags == 0, slot, R)
    uidx = jnp.zeros((R,), jnp.int32).at[tgt].set(
        jnp.arange(B * T, dtype=jnp.int32), mode="drop")
    out = _sc_combine(x_flat, final, idx_flat, uidx)
    return out.reshape(B, T, C)


# speedup vs baseline: 1.2221x; 1.0106x over previous
import functools
import math

import jax
import jax.numpy as jnp
from jax import lax
from jax.experimental import pallas as pl
from jax.experimental.pallas import tpu as pltpu
from jax.experimental.pallas import tpu_sc as plsc

B, T, C, H = 2, 2048, 2048, 16
DH = C // H
KCAP = T // 2
DFF = 4 * C
R = B * KCAP

_BF = jnp.bfloat16
_F32 = jnp.float32


_NC, _NS = 2, 16
_NW = _NC * _NS


def _sc_gather(table, idxs, n_rows, chunk=32):
    d = table.shape[1]
    per_w = n_rows // _NW
    n_chunks = per_w // chunk
    mesh = plsc.VectorSubcoreMesh(
        core_axis_name="c", subcore_axis_name="s",
        num_cores=_NC, num_subcores=_NS)

    @functools.partial(
        pl.kernel,
        out_type=jax.ShapeDtypeStruct((n_rows, d), table.dtype),
        mesh=mesh,
        scratch_types=[
            pltpu.VMEM((chunk,), jnp.int32),
            pltpu.VMEM((chunk, d), table.dtype),
            pltpu.SemaphoreType.DMA,
        ],
    )
    def k(table_hbm, idx_hbm, out_hbm, idx_v, rows_v, sem):
        wid = lax.axis_index("s") * _NC + lax.axis_index("c")
        for j in range(n_chunks):
            base = wid * per_w + j * chunk
            pltpu.sync_copy(idx_hbm.at[pl.ds(base, chunk)], idx_v)
            pltpu.async_copy(table_hbm.at[idx_v], rows_v, sem).wait()
            pltpu.sync_copy(rows_v, out_hbm.at[pl.ds(base, chunk)])

    return k(table, idxs)


def _sc_combine(x_flat, final, sidx, uidx):
    chunk = 32
    per_w = R // _NW
    n_chunks = per_w // chunk
    mesh = plsc.VectorSubcoreMesh(
        core_axis_name="c", subcore_axis_name="s",
        num_cores=_NC, num_subcores=_NS)

    @functools.partial(
        pl.kernel,
        out_type=jax.ShapeDtypeStruct((B * T, C), jnp.float32),
        mesh=mesh,
        scratch_types=[
            pltpu.VMEM((chunk,), jnp.int32),
            pltpu.VMEM((chunk, C), jnp.float32),
            pltpu.SemaphoreType.DMA,
        ],
    )
    def k(x_hbm, final_hbm, sidx_hbm, uidx_hbm, out_hbm, idx_v, rows_v, sem):
        wid = lax.axis_index("s") * _NC + lax.axis_index("c")
        for j in range(n_chunks):
            base = wid * per_w + j * chunk
            pltpu.sync_copy(sidx_hbm.at[pl.ds(base, chunk)], idx_v)
            pltpu.sync_copy(final_hbm.at[pl.ds(base, chunk)], rows_v)
            pltpu.async_copy(rows_v, out_hbm.at[idx_v], sem).wait()
        for j in range(n_chunks):
            base = wid * per_w + j * chunk
            pltpu.sync_copy(uidx_hbm.at[pl.ds(base, chunk)], idx_v)
            pltpu.async_copy(x_hbm.at[idx_v], rows_v, sem).wait()
            pltpu.async_copy(rows_v, out_hbm.at[idx_v], sem).wait()

    return k(x_flat, final, sidx, uidx)


def _router_body(x_ref, w_ref, o_ref):
    i = pl.program_id(0)
    lg = lax.dot_general(x_ref[...].astype(_BF), w_ref[...].astype(_BF),
                         (((1,), (0,)), ((), ())),
                         preferred_element_type=_F32)
    o_ref[pl.ds(i * 512, 512)] = lg[:, 0]


def _router_logits(x_flat, w_router):
    return pl.pallas_call(
        _router_body,
        grid=(B * T // 512,),
        in_specs=[
            pl.BlockSpec((512, C), lambda i: (i, 0)),
            pl.BlockSpec((C, 1), lambda i: (0, 0)),
        ],
        out_specs=pl.BlockSpec((B * T,), lambda i: (0,)),
        out_shape=jax.ShapeDtypeStruct((B * T,), _F32),
    )(x_flat, w_router.reshape(C, 1))


def _rms_body(x_ref, w_ref, o_ref):
    x = x_ref[...]
    ms = jnp.mean(x * x, axis=-1, keepdims=True)
    o_ref[...] = (x * lax.rsqrt(ms + 1e-6) * w_ref[...][None, :]).astype(_BF)


def _rmsnorm_bf16(x, w):
    return pl.pallas_call(
        _rms_body,
        grid=(R // 256,),
        in_specs=[
            pl.BlockSpec((256, C), lambda i: (i, 0)),
            pl.BlockSpec((C,), lambda i: (0,)),
        ],
        out_specs=pl.BlockSpec((256, C), lambda i: (i, 0)),
        out_shape=jax.ShapeDtypeStruct((R, C), _BF),
    )(x, w)


def _qkv_body(h_ref, wq_ref, wk_ref, wv_ref, q_ref, k_ref, v_ref):
    h = h_ref[...]
    for wref, oref in ((wq_ref, q_ref), (wk_ref, k_ref), (wv_ref, v_ref)):
        w = wref[...].astype(_BF)
        acc = lax.dot_general(h, w, (((1,), (0,)), ((), ())),
                              preferred_element_type=_F32)
        oref[...] = acc.astype(_BF)


def _qkv(h, Wq, Wk, Wv):
    bn = 512
    wspec = pl.BlockSpec((C, bn), lambda n: (0, n))
    ospec = pl.BlockSpec((R, bn), lambda n: (0, n))
    oshape = jax.ShapeDtypeStruct((R, C), _BF)
    return pl.pallas_call(
        _qkv_body,
        grid=(C // bn,),
        in_specs=[pl.BlockSpec((R, C), lambda n: (0, 0)), wspec, wspec, wspec],
        out_specs=(ospec, ospec, ospec),
        out_shape=(oshape, oshape, oshape),
    )(h, Wq, Wk, Wv)


def _att_body(q_ref, k_ref, v_ref, o_ref):
    q = q_ref[...]
    s = lax.dot_general(q, k_ref[...], (((1,), (1,)), ((), ())),
                        preferred_element_type=_F32)
    s = s * (1.0 / math.sqrt(DH))
    row = lax.broadcasted_iota(jnp.int32, (KCAP, KCAP), 0)
    col = lax.broadcasted_iota(jnp.int32, (KCAP, KCAP), 1)
    s = jnp.where(row >= col, s, -jnp.finfo(_F32).max)
    m = jnp.max(s, axis=-1, keepdims=True)
    e = jnp.exp(s - m)
    l = jnp.sum(e, axis=-1, keepdims=True)
    y = lax.dot_general(e.astype(_BF), v_ref[...], (((1,), (0,)), ((), ())),
                        preferred_element_type=_F32)
    o_ref[...] = (y / l).astype(_BF)


def _attention(q, k, v):
    spec = pl.BlockSpec((KCAP, DH), lambda b, h: (b, h))
    return pl.pallas_call(
        _att_body,
        grid=(B, H),
        in_specs=[spec, spec, spec],
        out_specs=spec,
        out_shape=jax.ShapeDtypeStruct((R, C), _BF),
    )(q, k, v)


def _wo_body(a_ref, wo_ref, sel_ref, gl_ref, ln2_ref, h2_ref, part_ref, acc_ref):
    kk = pl.program_id(1)
    nk = pl.num_programs(1)

    @pl.when(kk == 0)
    def _():
        acc_ref[...] = jnp.zeros_like(acc_ref)

    acc_ref[...] += lax.dot_general(
        a_ref[...], wo_ref[...].astype(_BF), (((1,), (0,)), ((), ())),
        preferred_element_type=_F32)

    @pl.when(kk == nk - 1)
    def _():
        y2 = acc_ref[...]
        selb = sel_ref[...]
        sel2 = selb + y2
        ms = jnp.mean(sel2 * sel2, axis=-1, keepdims=True)
        h2_ref[...] = (sel2 * lax.rsqrt(ms + 1e-6) * ln2_ref[...][None, :]).astype(_BF)
        gate = jax.nn.sigmoid(gl_ref[...])[:, None]
        part_ref[...] = selb + gate * y2


def _wo_norm(attnout, Wo, sel, gl, ln2_w):
    bm, bk = 1024, 256
    return pl.pallas_call(
        _wo_body,
        grid=(R // bm, C // bk),
        in_specs=[
            pl.BlockSpec((bm, bk), lambda m, k: (m, k)),
            pl.BlockSpec((bk, C), lambda m, k: (k, 0)),
            pl.BlockSpec((bm, C), lambda m, k: (m, 0)),
            pl.BlockSpec((bm,), lambda m, k: (m,)),
            pl.BlockSpec((C,), lambda m, k: (0,)),
        ],
        out_specs=(
            pl.BlockSpec((bm, C), lambda m, k: (m, 0)),
            pl.BlockSpec((bm, C), lambda m, k: (m, 0)),
        ),
        out_shape=(
            jax.ShapeDtypeStruct((R, C), _BF),
            jax.ShapeDtypeStruct((R, C), _F32),
        ),
        scratch_shapes=[pltpu.VMEM((bm, C), _F32)],
    )(attnout, Wo, sel, gl, ln2_w)


def _mlp1_body(h_ref, w_ref, o_ref):
    acc = lax.dot_general(h_ref[...], w_ref[...].astype(_BF),
                          (((1,), (0,)), ((), ())), preferred_element_type=_F32)
    o_ref[...] = jax.nn.gelu(acc, approximate=True).astype(_BF)


def _mlp1(h2, W1):
    bn = 1024
    return pl.pallas_call(
        _mlp1_body,
        grid=(DFF // bn,),
        in_specs=[
            pl.BlockSpec((R, C), lambda n: (0, 0)),
            pl.BlockSpec((C, bn), lambda n: (0, n)),
        ],
        out_specs=pl.BlockSpec((R, bn), lambda n: (0, n)),
        out_shape=jax.ShapeDtypeStruct((R, DFF), _BF),
    )(h2, W1)


def _mlp2_body(g_ref, w_ref, part_ref, gl_ref, o_ref, acc_ref):
    kk = pl.program_id(1)
    nk = pl.num_programs(1)

    @pl.when(kk == 0)
    def _():
        acc_ref[...] = jnp.zeros_like(acc_ref)

    acc_ref[...] += lax.dot_general(
        g_ref[...], w_ref[...].astype(_BF), (((1,), (0,)), ((), ())),
        preferred_element_type=_F32)

    @pl.when(kk == nk - 1)
    def _():
        gate = jax.nn.sigmoid(gl_ref[...])[:, None]
        o_ref[...] = part_ref[...] + gate * acc_ref[...]


def _mlp2_final(g, W2, partial, gl):
    bn, bk = 1024, 512
    return pl.pallas_call(
        _mlp2_body,
        grid=(C // bn, DFF // bk),
        in_specs=[
            pl.BlockSpec((R, bk), lambda n, k: (0, k)),
            pl.BlockSpec((bk, bn), lambda n, k: (k, n)),
            pl.BlockSpec((R, bn), lambda n, k: (0, n)),
            pl.BlockSpec((R,), lambda n, k: (0,)),
        ],
        out_specs=pl.BlockSpec((R, bn), lambda n, k: (0, n)),
        out_shape=jax.ShapeDtypeStruct((R, C), _F32),
        scratch_shapes=[pltpu.VMEM((R, bn), _F32)],
    )(g, W2, partial, gl)


def kernel(x, w_router, ln1_w, ln2_w, Wq, Wk, Wv, Wo, W1, W2):
    x_flat = x.reshape(B * T, C)

    logits = _router_logits(x_flat, w_router).reshape(B, T)
    _, idx = lax.top_k(logits, KCAP)
    idx = jnp.sort(idx, axis=1)
    gl = jnp.take_along_axis(logits, idx, axis=1).reshape(R)
    idx_flat = (idx + jnp.arange(B, dtype=idx.dtype)[:, None] * T).reshape(R)
    idx_flat = idx_flat.astype(jnp.int32)

    sel = _sc_gather(x_flat, idx_flat, R)

    h1 = _rmsnorm_bf16(sel, ln1_w)
    q, k, v = _qkv(h1, Wq, Wk, Wv)
    attnout = _attention(q, k, v)
    h2, partial = _wo_norm(attnout, Wo, sel, gl, ln2_w)
    g = _mlp1(h2, W1)
    final = _mlp2_final(g, W2, partial, gl)

    flags = jnp.zeros((B * T,), jnp.int32).at[idx_flat].set(1)
    slot = jnp.cumsum(1 - flags) - 1
    tgt = jnp.where(flags == 0, slot, R)
    uidx = jnp.zeros((R,), jnp.int32).at[tgt].set(
        jnp.arange(B * T, dtype=jnp.int32), mode="drop")
    out = _sc_combine(x_flat, final, idx_flat, uidx)
    return out.reshape(B, T, C)


# speedup vs baseline: 1.2702x; 1.0393x over previous
import functools
import math

import jax
import jax.numpy as jnp
from jax import lax
from jax.experimental import pallas as pl
from jax.experimental.pallas import tpu as pltpu
from jax.experimental.pallas import tpu_sc as plsc

B, T, C, H = 2, 2048, 2048, 16
DH = C // H
KCAP = T // 2
DFF = 4 * C
R = B * KCAP

_BF = jnp.bfloat16
_F32 = jnp.float32


_NC, _NS = 2, 16
_NW = _NC * _NS


def _sc_gather(table, idxs, n_rows, chunk=32):
    d = table.shape[1]
    per_w = n_rows // _NW
    n_chunks = per_w // chunk
    mesh = plsc.VectorSubcoreMesh(
        core_axis_name="c", subcore_axis_name="s",
        num_cores=_NC, num_subcores=_NS)

    @functools.partial(
        pl.kernel,
        out_type=jax.ShapeDtypeStruct((n_rows, d), table.dtype),
        mesh=mesh,
        scratch_types=[
            pltpu.VMEM((chunk,), jnp.int32),
            pltpu.VMEM((chunk, d), table.dtype),
            pltpu.SemaphoreType.DMA,
        ],
    )
    def k(table_hbm, idx_hbm, out_hbm, idx_v, rows_v, sem):
        wid = lax.axis_index("s") * _NC + lax.axis_index("c")
        for j in range(n_chunks):
            base = wid * per_w + j * chunk
            pltpu.sync_copy(idx_hbm.at[pl.ds(base, chunk)], idx_v)
            pltpu.async_copy(table_hbm.at[idx_v], rows_v, sem).wait()
            pltpu.sync_copy(rows_v, out_hbm.at[pl.ds(base, chunk)])

    return k(table, idxs)


def _sc_combine(x_flat, final, sidx, uidx):
    chunk = 32
    per_w = R // _NW
    n_chunks = per_w // chunk
    mesh = plsc.VectorSubcoreMesh(
        core_axis_name="c", subcore_axis_name="s",
        num_cores=_NC, num_subcores=_NS)

    @functools.partial(
        pl.kernel,
        out_type=jax.ShapeDtypeStruct((B * T, C), jnp.float32),
        mesh=mesh,
        scratch_types=[
            pltpu.VMEM((chunk,), jnp.int32),
            pltpu.VMEM((chunk, C), jnp.float32),
            pltpu.SemaphoreType.DMA,
        ],
    )
    def k(x_hbm, final_hbm, sidx_hbm, uidx_hbm, out_hbm, idx_v, rows_v, sem):
        wid = lax.axis_index("s") * _NC + lax.axis_index("c")
        for j in range(n_chunks):
            base = wid * per_w + j * chunk
            pltpu.sync_copy(sidx_hbm.at[pl.ds(base, chunk)], idx_v)
            pltpu.sync_copy(final_hbm.at[pl.ds(base, chunk)], rows_v)
            pltpu.async_copy(rows_v, out_hbm.at[idx_v], sem).wait()
        for j in range(n_chunks):
            base = wid * per_w + j * chunk
            pltpu.sync_copy(uidx_hbm.at[pl.ds(base, chunk)], idx_v)
            pltpu.async_copy(x_hbm.at[idx_v], rows_v, sem).wait()
            pltpu.async_copy(rows_v, out_hbm.at[idx_v], sem).wait()

    return k(x_flat, final, sidx, uidx)


def _router_body(x_ref, w_ref, o_ref):
    i = pl.program_id(0)
    lg = lax.dot_general(x_ref[...].astype(_BF), w_ref[...].astype(_BF),
                         (((1,), (0,)), ((), ())),
                         preferred_element_type=_F32)
    o_ref[pl.ds(i * 512, 512)] = lg[:, 0]


def _router_logits(x_flat, w_router):
    return pl.pallas_call(
        _router_body,
        grid=(B * T // 512,),
        in_specs=[
            pl.BlockSpec((512, C), lambda i: (i, 0)),
            pl.BlockSpec((C, 1), lambda i: (0, 0)),
        ],
        out_specs=pl.BlockSpec((B * T,), lambda i: (0,)),
        out_shape=jax.ShapeDtypeStruct((B * T,), _F32),
    )(x_flat, w_router.reshape(C, 1))


def _rms_body(x_ref, w_ref, o_ref):
    x = x_ref[...]
    ms = jnp.mean(x * x, axis=-1, keepdims=True)
    o_ref[...] = (x * lax.rsqrt(ms + 1e-6) * w_ref[...][None, :]).astype(_BF)


def _rmsnorm_bf16(x, w):
    return pl.pallas_call(
        _rms_body,
        grid=(R // 256,),
        in_specs=[
            pl.BlockSpec((256, C), lambda i: (i, 0)),
            pl.BlockSpec((C,), lambda i: (0,)),
        ],
        out_specs=pl.BlockSpec((256, C), lambda i: (i, 0)),
        out_shape=jax.ShapeDtypeStruct((R, C), _BF),
    )(x, w)


def _qkv_body(h_ref, wq_ref, wk_ref, wv_ref, q_ref, k_ref, v_ref):
    h = h_ref[...]
    for wref, oref in ((wq_ref, q_ref), (wk_ref, k_ref), (wv_ref, v_ref)):
        w = wref[...].astype(_BF)
        acc = lax.dot_general(h, w, (((1,), (0,)), ((), ())),
                              preferred_element_type=_F32)
        oref[...] = acc.astype(_BF)


def _qkv(h, Wq, Wk, Wv):
    bn = 512
    wspec = pl.BlockSpec((C, bn), lambda n: (0, n))
    ospec = pl.BlockSpec((R, bn), lambda n: (0, n))
    oshape = jax.ShapeDtypeStruct((R, C), _BF)
    return pl.pallas_call(
        _qkv_body,
        grid=(C // bn,),
        in_specs=[pl.BlockSpec((R, C), lambda n: (0, 0)), wspec, wspec, wspec],
        out_specs=(ospec, ospec, ospec),
        out_shape=(oshape, oshape, oshape),
    )(h, Wq, Wk, Wv)


def _att_body(q_ref, k_ref, v_ref, o_ref):
    row = lax.broadcasted_iota(jnp.int32, (KCAP, KCAP), 0)
    col = lax.broadcasted_iota(jnp.int32, (KCAP, KCAP), 1)
    causal = row >= col
    for hh in range(2):
        sl = pl.ds(hh * DH, DH)
        s = lax.dot_general(q_ref[:, sl], k_ref[:, sl],
                            (((1,), (1,)), ((), ())),
                            preferred_element_type=_F32)
        s = s * (1.0 / math.sqrt(DH))
        s = jnp.where(causal, s, -jnp.finfo(_F32).max)
        m = jnp.max(s, axis=-1, keepdims=True)
        e = jnp.exp(s - m)
        l = jnp.sum(e, axis=-1, keepdims=True)
        y = lax.dot_general(e.astype(_BF), v_ref[:, sl],
                            (((1,), (0,)), ((), ())),
                            preferred_element_type=_F32)
        o_ref[:, sl] = (y / l).astype(_BF)


def _attention(q, k, v):
    spec = pl.BlockSpec((KCAP, 2 * DH), lambda b, h: (b, h))
    return pl.pallas_call(
        _att_body,
        grid=(B, H // 2),
        in_specs=[spec, spec, spec],
        out_specs=spec,
        out_shape=jax.ShapeDtypeStruct((R, C), _BF),
    )(q, k, v)


def _wo_body(a_ref, wo_ref, sel_ref, gl_ref, ln2_ref, h2_ref, part_ref, acc_ref):
    kk = pl.program_id(1)
    nk = pl.num_programs(1)

    @pl.when(kk == 0)
    def _():
        acc_ref[...] = jnp.zeros_like(acc_ref)

    acc_ref[...] += lax.dot_general(
        a_ref[...], wo_ref[...].astype(_BF), (((1,), (0,)), ((), ())),
        preferred_element_type=_F32)

    @pl.when(kk == nk - 1)
    def _():
        y2 = acc_ref[...]
        selb = sel_ref[...]
        sel2 = selb + y2
        ms = jnp.mean(sel2 * sel2, axis=-1, keepdims=True)
        h2_ref[...] = (sel2 * lax.rsqrt(ms + 1e-6) * ln2_ref[...][None, :]).astype(_BF)
        gate = jax.nn.sigmoid(gl_ref[...])[:, None]
        part_ref[...] = selb + gate * y2


def _wo_norm(attnout, Wo, sel, gl, ln2_w):
    bm, bk = 1024, 256
    return pl.pallas_call(
        _wo_body,
        grid=(R // bm, C // bk),
        in_specs=[
            pl.BlockSpec((bm, bk), lambda m, k: (m, k)),
            pl.BlockSpec((bk, C), lambda m, k: (k, 0)),
            pl.BlockSpec((bm, C), lambda m, k: (m, 0)),
            pl.BlockSpec((bm,), lambda m, k: (m,)),
            pl.BlockSpec((C,), lambda m, k: (0,)),
        ],
        out_specs=(
            pl.BlockSpec((bm, C), lambda m, k: (m, 0)),
            pl.BlockSpec((bm, C), lambda m, k: (m, 0)),
        ),
        out_shape=(
            jax.ShapeDtypeStruct((R, C), _BF),
            jax.ShapeDtypeStruct((R, C), _F32),
        ),
        scratch_shapes=[pltpu.VMEM((bm, C), _F32)],
    )(attnout, Wo, sel, gl, ln2_w)


def _mlp1_body(h_ref, w_ref, o_ref):
    acc = lax.dot_general(h_ref[...], w_ref[...].astype(_BF),
                          (((1,), (0,)), ((), ())), preferred_element_type=_F32)
    o_ref[...] = jax.nn.gelu(acc, approximate=True).astype(_BF)


def _mlp1(h2, W1):
    bn = 1024
    return pl.pallas_call(
        _mlp1_body,
        grid=(DFF // bn,),
        in_specs=[
            pl.BlockSpec((R, C), lambda n: (0, 0)),
            pl.BlockSpec((C, bn), lambda n: (0, n)),
        ],
        out_specs=pl.BlockSpec((R, bn), lambda n: (0, n)),
        out_shape=jax.ShapeDtypeStruct((R, DFF), _BF),
    )(h2, W1)


def _mlp2_body(g_ref, w_ref, part_ref, gl_ref, o_ref, acc_ref):
    kk = pl.program_id(1)
    nk = pl.num_programs(1)

    @pl.when(kk == 0)
    def _():
        acc_ref[...] = jnp.zeros_like(acc_ref)

    acc_ref[...] += lax.dot_general(
        g_ref[...], w_ref[...].astype(_BF), (((1,), (0,)), ((), ())),
        preferred_element_type=_F32)

    @pl.when(kk == nk - 1)
    def _():
        gate = jax.nn.sigmoid(gl_ref[...])[:, None]
        o_ref[...] = part_ref[...] + gate * acc_ref[...]


def _mlp2_final(g, W2, partial, gl):
    bn, bk = 1024, 512
    return pl.pallas_call(
        _mlp2_body,
        grid=(C // bn, DFF // bk),
        in_specs=[
            pl.BlockSpec((R, bk), lambda n, k: (0, k)),
            pl.BlockSpec((bk, bn), lambda n, k: (k, n)),
            pl.BlockSpec((R, bn), lambda n, k: (0, n)),
            pl.BlockSpec((R,), lambda n, k: (0,)),
        ],
        out_specs=pl.BlockSpec((R, bn), lambda n, k: (0, n)),
        out_shape=jax.ShapeDtypeStruct((R, C), _F32),
        scratch_shapes=[pltpu.VMEM((R, bn), _F32)],
    )(g, W2, partial, gl)


def kernel(x, w_router, ln1_w, ln2_w, Wq, Wk, Wv, Wo, W1, W2):
    x_flat = x.reshape(B * T, C)

    logits = _router_logits(x_flat, w_router).reshape(B, T)
    _, idx = lax.top_k(logits, KCAP)
    idx = jnp.sort(idx, axis=1)
    gl = jnp.take_along_axis(logits, idx, axis=1).reshape(R)
    idx_flat = (idx + jnp.arange(B, dtype=idx.dtype)[:, None] * T).reshape(R)
    idx_flat = idx_flat.astype(jnp.int32)

    sel = _sc_gather(x_flat, idx_flat, R)

    h1 = _rmsnorm_bf16(sel, ln1_w)
    q, k, v = _qkv(h1, Wq, Wk, Wv)
    attnout = _attention(q, k, v)
    h2, partial = _wo_norm(attnout, Wo, sel, gl, ln2_w)
    g = _mlp1(h2, W1)
    final = _mlp2_final(g, W2, partial, gl)

    flags = jnp.zeros((B * T,), jnp.int32).at[idx_flat].set(1)
    slot = jnp.cumsum(1 - flags) - 1
    tgt = jnp.where(flags == 0, slot, R)
    uidx = jnp.zeros((R,), jnp.int32).at[tgt].set(
        jnp.arange(B * T, dtype=jnp.int32), mode="drop")
    out = _sc_combine(x_flat, final, idx_flat, uidx)
    return out.reshape(B, T, C)


# speedup vs baseline: 1.2954x; 1.0199x over previous
import functools
import math

import jax
import jax.numpy as jnp
from jax import lax
from jax.experimental import pallas as pl
from jax.experimental.pallas import tpu as pltpu
from jax.experimental.pallas import tpu_sc as plsc

B, T, C, H = 2, 2048, 2048, 16
DH = C // H
KCAP = T // 2
DFF = 4 * C
R = B * KCAP

_BF = jnp.bfloat16
_F32 = jnp.float32


_NC, _NS = 2, 16
_NW = _NC * _NS


def _sc_gather(table, idxs, n_rows, chunk=32):
    d = table.shape[1]
    per_w = n_rows // _NW
    n_chunks = per_w // chunk
    mesh = plsc.VectorSubcoreMesh(
        core_axis_name="c", subcore_axis_name="s",
        num_cores=_NC, num_subcores=_NS)

    @functools.partial(
        pl.kernel,
        out_type=jax.ShapeDtypeStruct((n_rows, d), table.dtype),
        mesh=mesh,
        scratch_types=[
            pltpu.VMEM((chunk,), jnp.int32),
            pltpu.VMEM((chunk, d), table.dtype),
            pltpu.SemaphoreType.DMA,
        ],
    )
    def k(table_hbm, idx_hbm, out_hbm, idx_v, rows_v, sem):
        wid = lax.axis_index("s") * _NC + lax.axis_index("c")
        for j in range(n_chunks):
            base = wid * per_w + j * chunk
            pltpu.sync_copy(idx_hbm.at[pl.ds(base, chunk)], idx_v)
            pltpu.async_copy(table_hbm.at[idx_v], rows_v, sem).wait()
            pltpu.sync_copy(rows_v, out_hbm.at[pl.ds(base, chunk)])

    return k(table, idxs)


def _sc_combine(x_flat, final, sidx, uidx):
    chunk = 32
    per_w = R // _NW
    n_chunks = per_w // chunk
    mesh = plsc.VectorSubcoreMesh(
        core_axis_name="c", subcore_axis_name="s",
        num_cores=_NC, num_subcores=_NS)

    @functools.partial(
        pl.kernel,
        out_type=jax.ShapeDtypeStruct((B * T, C), jnp.float32),
        mesh=mesh,
        scratch_types=[
            pltpu.VMEM((chunk,), jnp.int32),
            pltpu.VMEM((chunk, C), jnp.float32),
            pltpu.SemaphoreType.DMA,
        ],
    )
    def k(x_hbm, final_hbm, sidx_hbm, uidx_hbm, out_hbm, idx_v, rows_v, sem):
        wid = lax.axis_index("s") * _NC + lax.axis_index("c")
        for j in range(n_chunks):
            base = wid * per_w + j * chunk
            pltpu.sync_copy(sidx_hbm.at[pl.ds(base, chunk)], idx_v)
            pltpu.sync_copy(final_hbm.at[pl.ds(base, chunk)], rows_v)
            pltpu.async_copy(rows_v, out_hbm.at[idx_v], sem).wait()
        for j in range(n_chunks):
            base = wid * per_w + j * chunk
            pltpu.sync_copy(uidx_hbm.at[pl.ds(base, chunk)], idx_v)
            pltpu.async_copy(x_hbm.at[idx_v], rows_v, sem).wait()
            pltpu.async_copy(rows_v, out_hbm.at[idx_v], sem).wait()

    return k(x_flat, final, sidx, uidx)


def _router_body(x_ref, w_ref, o_ref):
    i = pl.program_id(0)
    lg = lax.dot_general(x_ref[...].astype(_BF), w_ref[...].astype(_BF),
                         (((1,), (0,)), ((), ())),
                         preferred_element_type=_F32)
    o_ref[pl.ds(i * 512, 512)] = lg[:, 0]


def _router_logits(x_flat, w_router):
    return pl.pallas_call(
        _router_body,
        grid=(B * T // 512,),
        in_specs=[
            pl.BlockSpec((512, C), lambda i: (i, 0)),
            pl.BlockSpec((C, 1), lambda i: (0, 0)),
        ],
        out_specs=pl.BlockSpec((B * T,), lambda i: (0,)),
        out_shape=jax.ShapeDtypeStruct((B * T,), _F32),
    )(x_flat, w_router.reshape(C, 1))


def _rms_body(x_ref, w_ref, o_ref):
    x = x_ref[...]
    ms = jnp.mean(x * x, axis=-1, keepdims=True)
    o_ref[...] = (x * lax.rsqrt(ms + 1e-6) * w_ref[...][None, :]).astype(_BF)


def _rmsnorm_bf16(x, w):
    return pl.pallas_call(
        _rms_body,
        grid=(R // 256,),
        in_specs=[
            pl.BlockSpec((256, C), lambda i: (i, 0)),
            pl.BlockSpec((C,), lambda i: (0,)),
        ],
        out_specs=pl.BlockSpec((256, C), lambda i: (i, 0)),
        out_shape=jax.ShapeDtypeStruct((R, C), _BF),
    )(x, w)


def _qkv_body(h_ref, wq_ref, wk_ref, wv_ref, q_ref, k_ref, v_ref):
    h = h_ref[...]
    for wref, oref in ((wq_ref, q_ref), (wk_ref, k_ref), (wv_ref, v_ref)):
        w = wref[...].astype(_BF)
        acc = lax.dot_general(h, w, (((1,), (0,)), ((), ())),
                              preferred_element_type=_F32)
        oref[...] = acc.astype(_BF)


def _qkv(h, Wq, Wk, Wv):
    bn = 512
    wspec = pl.BlockSpec((C, bn), lambda n: (0, n))
    ospec = pl.BlockSpec((R, bn), lambda n: (0, n))
    oshape = jax.ShapeDtypeStruct((R, C), _BF)
    return pl.pallas_call(
        _qkv_body,
        grid=(C // bn,),
        in_specs=[pl.BlockSpec((R, C), lambda n: (0, 0)), wspec, wspec, wspec],
        out_specs=(ospec, ospec, ospec),
        out_shape=(oshape, oshape, oshape),
    )(h, Wq, Wk, Wv)


def _att_body(q_ref, k_ref, v_ref, o_ref):
    row = lax.broadcasted_iota(jnp.int32, (KCAP, KCAP), 0)
    col = lax.broadcasted_iota(jnp.int32, (KCAP, KCAP), 1)
    causal = row >= col
    for hh in range(4):
        sl = pl.ds(hh * DH, DH)
        s = lax.dot_general(q_ref[:, sl], k_ref[:, sl],
                            (((1,), (1,)), ((), ())),
                            preferred_element_type=_F32)
        s = s * (1.0 / math.sqrt(DH))
        s = jnp.where(causal, s, -jnp.finfo(_F32).max)
        m = jnp.max(s, axis=-1, keepdims=True)
        e = jnp.exp(s - m)
        l = jnp.sum(e, axis=-1, keepdims=True)
        y = lax.dot_general(e.astype(_BF), v_ref[:, sl],
                            (((1,), (0,)), ((), ())),
                            preferred_element_type=_F32)
        o_ref[:, sl] = (y / l).astype(_BF)


def _attention(q, k, v):
    spec = pl.BlockSpec((KCAP, 4 * DH), lambda b, h: (b, h))
    return pl.pallas_call(
        _att_body,
        grid=(B, H // 4),
        in_specs=[spec, spec, spec],
        out_specs=spec,
        out_shape=jax.ShapeDtypeStruct((R, C), _BF),
    )(q, k, v)


def _wo_body(a_ref, wo_ref, sel_ref, gl_ref, ln2_ref, h2_ref, part_ref, acc_ref):
    kk = pl.program_id(1)
    nk = pl.num_programs(1)

    @pl.when(kk == 0)
    def _():
        acc_ref[...] = jnp.zeros_like(acc_ref)

    acc_ref[...] += lax.dot_general(
        a_ref[...], wo_ref[...].astype(_BF), (((1,), (0,)), ((), ())),
        preferred_element_type=_F32)

    @pl.when(kk == nk - 1)
    def _():
        y2 = acc_ref[...]
        selb = sel_ref[...]
        sel2 = selb + y2
        ms = jnp.mean(sel2 * sel2, axis=-1, keepdims=True)
        h2_ref[...] = (sel2 * lax.rsqrt(ms + 1e-6) * ln2_ref[...][None, :]).astype(_BF)
        gate = jax.nn.sigmoid(gl_ref[...])[:, None]
        part_ref[...] = selb + gate * y2


def _wo_norm(attnout, Wo, sel, gl, ln2_w):
    bm, bk = 1024, 256
    return pl.pallas_call(
        _wo_body,
        grid=(R // bm, C // bk),
        in_specs=[
            pl.BlockSpec((bm, bk), lambda m, k: (m, k)),
            pl.BlockSpec((bk, C), lambda m, k: (k, 0)),
            pl.BlockSpec((bm, C), lambda m, k: (m, 0)),
            pl.BlockSpec((bm,), lambda m, k: (m,)),
            pl.BlockSpec((C,), lambda m, k: (0,)),
        ],
        out_specs=(
            pl.BlockSpec((bm, C), lambda m, k: (m, 0)),
            pl.BlockSpec((bm, C), lambda m, k: (m, 0)),
        ),
        out_shape=(
            jax.ShapeDtypeStruct((R, C), _BF),
            jax.ShapeDtypeStruct((R, C), _F32),
        ),
        scratch_shapes=[pltpu.VMEM((bm, C), _F32)],
    )(attnout, Wo, sel, gl, ln2_w)


def _mlp1_body(h_ref, w_ref, o_ref):
    acc = lax.dot_general(h_ref[...], w_ref[...].astype(_BF),
                          (((1,), (0,)), ((), ())), preferred_element_type=_F32)
    o_ref[...] = jax.nn.gelu(acc, approximate=True).astype(_BF)


def _mlp1(h2, W1):
    bn = 1024
    return pl.pallas_call(
        _mlp1_body,
        grid=(DFF // bn,),
        in_specs=[
            pl.BlockSpec((R, C), lambda n: (0, 0)),
            pl.BlockSpec((C, bn), lambda n: (0, n)),
        ],
        out_specs=pl.BlockSpec((R, bn), lambda n: (0, n)),
        out_shape=jax.ShapeDtypeStruct((R, DFF), _BF),
    )(h2, W1)


def _mlp2_body(g_ref, w_ref, part_ref, gl_ref, o_ref, acc_ref):
    kk = pl.program_id(1)
    nk = pl.num_programs(1)

    @pl.when(kk == 0)
    def _():
        acc_ref[...] = jnp.zeros_like(acc_ref)

    acc_ref[...] += lax.dot_general(
        g_ref[...], w_ref[...].astype(_BF), (((1,), (0,)), ((), ())),
        preferred_element_type=_F32)

    @pl.when(kk == nk - 1)
    def _():
        gate = jax.nn.sigmoid(gl_ref[...])[:, None]
        o_ref[...] = part_ref[...] + gate * acc_ref[...]


def _mlp2_final(g, W2, partial, gl):
    bn, bk = 1024, 512
    return pl.pallas_call(
        _mlp2_body,
        grid=(C // bn, DFF // bk),
        in_specs=[
            pl.BlockSpec((R, bk), lambda n, k: (0, k)),
            pl.BlockSpec((bk, bn), lambda n, k: (k, n)),
            pl.BlockSpec((R, bn), lambda n, k: (0, n)),
            pl.BlockSpec((R,), lambda n, k: (0,)),
        ],
        out_specs=pl.BlockSpec((R, bn), lambda n, k: (0, n)),
        out_shape=jax.ShapeDtypeStruct((R, C), _F32),
        scratch_shapes=[pltpu.VMEM((R, bn), _F32)],
    )(g, W2, partial, gl)


def kernel(x, w_router, ln1_w, ln2_w, Wq, Wk, Wv, Wo, W1, W2):
    x_flat = x.reshape(B * T, C)

    logits = _router_logits(x_flat, w_router).reshape(B, T)
    _, idx = lax.top_k(logits, KCAP)
    idx = jnp.sort(idx, axis=1)
    gl = jnp.take_along_axis(logits, idx, axis=1).reshape(R)
    idx_flat = (idx + jnp.arange(B, dtype=idx.dtype)[:, None] * T).reshape(R)
    idx_flat = idx_flat.astype(jnp.int32)

    sel = _sc_gather(x_flat, idx_flat, R)

    h1 = _rmsnorm_bf16(sel, ln1_w)
    q, k, v = _qkv(h1, Wq, Wk, Wv)
    attnout = _attention(q, k, v)
    h2, partial = _wo_norm(attnout, Wo, sel, gl, ln2_w)
    g = _mlp1(h2, W1)
    final = _mlp2_final(g, W2, partial, gl)

    flags = jnp.zeros((B * T,), jnp.int32).at[idx_flat].set(1)
    slot = jnp.cumsum(1 - flags) - 1
    tgt = jnp.where(flags == 0, slot, R)
    uidx = jnp.zeros((R,), jnp.int32).at[tgt].set(
        jnp.arange(B * T, dtype=jnp.int32), mode="drop")
    out = _sc_combine(x_flat, final, idx_flat, uidx)
    return out.reshape(B, T, C)


# speedup vs baseline: 1.2999x; 1.0034x over previous
import functools
import math

import jax
import jax.numpy as jnp
from jax import lax
from jax.experimental import pallas as pl
from jax.experimental.pallas import tpu as pltpu
from jax.experimental.pallas import tpu_sc as plsc

B, T, C, H = 2, 2048, 2048, 16
DH = C // H
KCAP = T // 2
DFF = 4 * C
R = B * KCAP

_BF = jnp.bfloat16
_F32 = jnp.float32


_NC, _NS = 2, 16
_NW = _NC * _NS


def _sc_gather(table, idxs, n_rows, chunk=32):
    d = table.shape[1]
    per_w = n_rows // _NW
    n_chunks = per_w // chunk
    mesh = plsc.VectorSubcoreMesh(
        core_axis_name="c", subcore_axis_name="s",
        num_cores=_NC, num_subcores=_NS)

    @functools.partial(
        pl.kernel,
        out_type=jax.ShapeDtypeStruct((n_rows, d), table.dtype),
        mesh=mesh,
        scratch_types=[
            pltpu.VMEM((chunk,), jnp.int32),
            pltpu.VMEM((chunk, d), table.dtype),
            pltpu.SemaphoreType.DMA,
        ],
    )
    def k(table_hbm, idx_hbm, out_hbm, idx_v, rows_v, sem):
        wid = lax.axis_index("s") * _NC + lax.axis_index("c")
        for j in range(n_chunks):
            base = wid * per_w + j * chunk
            pltpu.sync_copy(idx_hbm.at[pl.ds(base, chunk)], idx_v)
            pltpu.async_copy(table_hbm.at[idx_v], rows_v, sem).wait()
            pltpu.sync_copy(rows_v, out_hbm.at[pl.ds(base, chunk)])

    return k(table, idxs)


def _sc_combine(x_flat, final, sidx, uidx):
    chunk = 32
    per_w = R // _NW
    n_chunks = per_w // chunk
    mesh = plsc.VectorSubcoreMesh(
        core_axis_name="c", subcore_axis_name="s",
        num_cores=_NC, num_subcores=_NS)

    @functools.partial(
        pl.kernel,
        out_type=jax.ShapeDtypeStruct((B * T, C), jnp.float32),
        mesh=mesh,
        scratch_types=[
            pltpu.VMEM((chunk,), jnp.int32),
            pltpu.VMEM((chunk, C), jnp.float32),
            pltpu.SemaphoreType.DMA,
        ],
    )
    def k(x_hbm, final_hbm, sidx_hbm, uidx_hbm, out_hbm, idx_v, rows_v, sem):
        wid = lax.axis_index("s") * _NC + lax.axis_index("c")
        for j in range(n_chunks):
            base = wid * per_w + j * chunk
            pltpu.sync_copy(sidx_hbm.at[pl.ds(base, chunk)], idx_v)
            pltpu.sync_copy(final_hbm.at[pl.ds(base, chunk)], rows_v)
            pltpu.async_copy(rows_v, out_hbm.at[idx_v], sem).wait()
        for j in range(n_chunks):
            base = wid * per_w + j * chunk
            pltpu.sync_copy(uidx_hbm.at[pl.ds(base, chunk)], idx_v)
            pltpu.async_copy(x_hbm.at[idx_v], rows_v, sem).wait()
            pltpu.async_copy(rows_v, out_hbm.at[idx_v], sem).wait()

    return k(x_flat, final, sidx, uidx)


def _router_body(x_ref, w_ref, o_ref):
    i = pl.program_id(0)
    lg = lax.dot_general(x_ref[...].astype(_BF), w_ref[...].astype(_BF),
                         (((1,), (0,)), ((), ())),
                         preferred_element_type=_F32)
    o_ref[pl.ds(i * 512, 512)] = lg[:, 0]


def _router_logits(x_flat, w_router):
    return pl.pallas_call(
        _router_body,
        grid=(B * T // 512,),
        in_specs=[
            pl.BlockSpec((512, C), lambda i: (i, 0)),
            pl.BlockSpec((C, 1), lambda i: (0, 0)),
        ],
        out_specs=pl.BlockSpec((B * T,), lambda i: (0,)),
        out_shape=jax.ShapeDtypeStruct((B * T,), _F32),
    )(x_flat, w_router.reshape(C, 1))


def _rms_body(x_ref, w_ref, o_ref):
    x = x_ref[...]
    ms = jnp.mean(x * x, axis=-1, keepdims=True)
    o_ref[...] = (x * lax.rsqrt(ms + 1e-6) * w_ref[...][None, :]).astype(_BF)


def _rmsnorm_bf16(x, w):
    return pl.pallas_call(
        _rms_body,
        grid=(R // 256,),
        in_specs=[
            pl.BlockSpec((256, C), lambda i: (i, 0)),
            pl.BlockSpec((C,), lambda i: (0,)),
        ],
        out_specs=pl.BlockSpec((256, C), lambda i: (i, 0)),
        out_shape=jax.ShapeDtypeStruct((R, C), _BF),
    )(x, w)


def _qkv_body(h_ref, wq_ref, wk_ref, wv_ref, q_ref, k_ref, v_ref):
    h = h_ref[...]
    for wref, oref in ((wq_ref, q_ref), (wk_ref, k_ref), (wv_ref, v_ref)):
        w = wref[...].astype(_BF)
        acc = lax.dot_general(h, w, (((1,), (0,)), ((), ())),
                              preferred_element_type=_F32)
        oref[...] = acc.astype(_BF)


def _qkv(h, Wq, Wk, Wv):
    bn = 512
    wspec = pl.BlockSpec((C, bn), lambda n: (0, n))
    ospec = pl.BlockSpec((R, bn), lambda n: (0, n))
    oshape = jax.ShapeDtypeStruct((R, C), _BF)
    return pl.pallas_call(
        _qkv_body,
        grid=(C // bn,),
        in_specs=[pl.BlockSpec((R, C), lambda n: (0, 0)), wspec, wspec, wspec],
        out_specs=(ospec, ospec, ospec),
        out_shape=(oshape, oshape, oshape),
    )(h, Wq, Wk, Wv)


def _att_body(q_ref, k_ref, v_ref, o_ref):
    row = lax.broadcasted_iota(jnp.int32, (KCAP, KCAP), 0)
    col = lax.broadcasted_iota(jnp.int32, (KCAP, KCAP), 1)
    causal = row >= col
    for hh in range(8):
        sl = pl.ds(hh * DH, DH)
        s = lax.dot_general(q_ref[:, sl], k_ref[:, sl],
                            (((1,), (1,)), ((), ())),
                            preferred_element_type=_F32)
        s = s * (1.0 / math.sqrt(DH))
        s = jnp.where(causal, s, -jnp.finfo(_F32).max)
        m = jnp.max(s, axis=-1, keepdims=True)
        e = jnp.exp(s - m)
        l = jnp.sum(e, axis=-1, keepdims=True)
        y = lax.dot_general(e.astype(_BF), v_ref[:, sl],
                            (((1,), (0,)), ((), ())),
                            preferred_element_type=_F32)
        o_ref[:, sl] = (y / l).astype(_BF)


def _attention(q, k, v):
    spec = pl.BlockSpec((KCAP, 8 * DH), lambda b, h: (b, h))
    return pl.pallas_call(
        _att_body,
        grid=(B, H // 8),
        in_specs=[spec, spec, spec],
        out_specs=spec,
        out_shape=jax.ShapeDtypeStruct((R, C), _BF),
    )(q, k, v)


def _wo_body(a_ref, wo_ref, sel_ref, gl_ref, ln2_ref, h2_ref, part_ref, acc_ref):
    kk = pl.program_id(1)
    nk = pl.num_programs(1)

    @pl.when(kk == 0)
    def _():
        acc_ref[...] = jnp.zeros_like(acc_ref)

    acc_ref[...] += lax.dot_general(
        a_ref[...], wo_ref[...].astype(_BF), (((1,), (0,)), ((), ())),
        preferred_element_type=_F32)

    @pl.when(kk == nk - 1)
    def _():
        y2 = acc_ref[...]
        selb = sel_ref[...]
        sel2 = selb + y2
        ms = jnp.mean(sel2 * sel2, axis=-1, keepdims=True)
        h2_ref[...] = (sel2 * lax.rsqrt(ms + 1e-6) * ln2_ref[...][None, :]).astype(_BF)
        gate = jax.nn.sigmoid(gl_ref[...])[:, None]
        part_ref[...] = selb + gate * y2


def _wo_norm(attnout, Wo, sel, gl, ln2_w):
    bm, bk = 1024, 256
    return pl.pallas_call(
        _wo_body,
        grid=(R // bm, C // bk),
        in_specs=[
            pl.BlockSpec((bm, bk), lambda m, k: (m, k)),
            pl.BlockSpec((bk, C), lambda m, k: (k, 0)),
            pl.BlockSpec((bm, C), lambda m, k: (m, 0)),
            pl.BlockSpec((bm,), lambda m, k: (m,)),
            pl.BlockSpec((C,), lambda m, k: (0,)),
        ],
        out_specs=(
            pl.BlockSpec((bm, C), lambda m, k: (m, 0)),
            pl.BlockSpec((bm, C), lambda m, k: (m, 0)),
        ),
        out_shape=(
            jax.ShapeDtypeStruct((R, C), _BF),
            jax.ShapeDtypeStruct((R, C), _F32),
        ),
        scratch_shapes=[pltpu.VMEM((bm, C), _F32)],
    )(attnout, Wo, sel, gl, ln2_w)


def _mlp1_body(h_ref, w_ref, o_ref):
    acc = lax.dot_general(h_ref[...], w_ref[...].astype(_BF),
                          (((1,), (0,)), ((), ())), preferred_element_type=_F32)
    o_ref[...] = jax.nn.gelu(acc, approximate=True).astype(_BF)


def _mlp1(h2, W1):
    bn = 1024
    return pl.pallas_call(
        _mlp1_body,
        grid=(DFF // bn,),
        in_specs=[
            pl.BlockSpec((R, C), lambda n: (0, 0)),
            pl.BlockSpec((C, bn), lambda n: (0, n)),
        ],
        out_specs=pl.BlockSpec((R, bn), lambda n: (0, n)),
        out_shape=jax.ShapeDtypeStruct((R, DFF), _BF),
    )(h2, W1)


def _mlp2_body(g_ref, w_ref, part_ref, gl_ref, o_ref, acc_ref):
    kk = pl.program_id(1)
    nk = pl.num_programs(1)

    @pl.when(kk == 0)
    def _():
        acc_ref[...] = jnp.zeros_like(acc_ref)

    acc_ref[...] += lax.dot_general(
        g_ref[...], w_ref[...].astype(_BF), (((1,), (0,)), ((), ())),
        preferred_element_type=_F32)

    @pl.when(kk == nk - 1)
    def _():
        gate = jax.nn.sigmoid(gl_ref[...])[:, None]
        o_ref[...] = part_ref[...] + gate * acc_ref[...]


def _mlp2_final(g, W2, partial, gl):
    bn, bk = 1024, 512
    return pl.pallas_call(
        _mlp2_body,
        grid=(C // bn, DFF // bk),
        in_specs=[
            pl.BlockSpec((R, bk), lambda n, k: (0, k)),
            pl.BlockSpec((bk, bn), lambda n, k: (k, n)),
            pl.BlockSpec((R, bn), lambda n, k: (0, n)),
            pl.BlockSpec((R,), lambda n, k: (0,)),
        ],
        out_specs=pl.BlockSpec((R, bn), lambda n, k: (0, n)),
        out_shape=jax.ShapeDtypeStruct((R, C), _F32),
        scratch_shapes=[pltpu.VMEM((R, bn), _F32)],
    )(g, W2, partial, gl)


def kernel(x, w_router, ln1_w, ln2_w, Wq, Wk, Wv, Wo, W1, W2):
    x_flat = x.reshape(B * T, C)

    logits = _router_logits(x_flat, w_router).reshape(B, T)
    _, idx = lax.top_k(logits, KCAP)
    idx = jnp.sort(idx, axis=1)
    gl = jnp.take_along_axis(logits, idx, axis=1).reshape(R)
    idx_flat = (idx + jnp.arange(B, dtype=idx.dtype)[:, None] * T).reshape(R)
    idx_flat = idx_flat.astype(jnp.int32)

    sel = _sc_gather(x_flat, idx_flat, R)

    h1 = _rmsnorm_bf16(sel, ln1_w)
    q, k, v = _qkv(h1, Wq, Wk, Wv)
    attnout = _attention(q, k, v)
    h2, partial = _wo_norm(attnout, Wo, sel, gl, ln2_w)
    g = _mlp1(h2, W1)
    final = _mlp2_final(g, W2, partial, gl)

    flags = jnp.zeros((B * T,), jnp.int32).at[idx_flat].set(1)
    slot = jnp.cumsum(1 - flags) - 1
    tgt = jnp.where(flags == 0, slot, R)
    uidx = jnp.zeros((R,), jnp.int32).at[tgt].set(
        jnp.arange(B * T, dtype=jnp.int32), mode="drop")
    out = _sc_combine(x_flat, final, idx_flat, uidx)
    return out.reshape(B, T, C)
